# baseline (speedup 1.0000x reference)
import jax
import jax.numpy as jnp
from jax.experimental import pallas as pl
from jax.experimental.pallas import tpu as pltpu

_NUM_EXPERTS = 8
_TOP_K = 2
_HIDDEN = 1024
_INTER = 2816
_NUM_TOKENS = 32

_BI = 1408
_NCHUNKS = _NUM_EXPERTS * 2
_NBUF = 3


def _issue(w13_hbm, w2_hbm, buf13, buf2, sem13, sem2, e, i, slot):
    pltpu.make_async_copy(
        w13_hbm.at[e, :, pl.ds(i * _BI, _BI), :],
        buf13.at[slot], sem13.at[slot]).start()
    pltpu.make_async_copy(
        w2_hbm.at[e, :, pl.ds(i * _BI, _BI)],
        buf2.at[slot], sem2.at[slot]).start()


def _moe_body(x_ref, rl_ref, w13_hbm, w2_hbm, out_ref,
              wte_ref, buf13, buf2, sem13, sem2):
    for c in range(_NBUF):
        _issue(w13_hbm, w2_hbm, buf13, buf2, sem13, sem2,
               c // 2, c % 2, c)

    logits = rl_ref[...]
    m = jnp.max(logits, axis=-1, keepdims=True)
    p = jnp.exp(logits - m)
    p = p / jnp.sum(p, axis=-1, keepdims=True)
    idx = jax.lax.broadcasted_iota(jnp.int32, p.shape, 1)
    m1 = jnp.max(p, axis=-1, keepdims=True)
    i1 = jnp.min(jnp.where(p == m1, idx, _NUM_EXPERTS), axis=-1,
                 keepdims=True)
    p2 = jnp.where(idx == i1, -jnp.inf, p)
    m2 = jnp.max(p2, axis=-1, keepdims=True)
    i2 = jnp.min(jnp.where(p2 == m2, idx, _NUM_EXPERTS), axis=-1,
                 keepdims=True)
    s = m1 + m2
    wte_ref[...] = jnp.where(idx == i1, m1, jnp.where(idx == i2, m2, 0.0)) / s
    out_ref[...] = jnp.zeros_like(out_ref)

    xb = x_ref[...].astype(jnp.bfloat16)
    eidx = jax.lax.broadcasted_iota(jnp.int32, (_NUM_TOKENS, _NUM_EXPERTS), 1)
    dims = (((1,), (1,)), ((), ()))

    def chunk_body(c, _):
        slot = jax.lax.rem(c, _NBUF)
        e = c // 2
        i = jax.lax.rem(c, 2)
        pltpu.make_async_copy(
            w13_hbm.at[e, :, pl.ds(i * _BI, _BI), :],
            buf13.at[slot], sem13.at[slot]).wait()
        gate_w = buf13[slot, 0].astype(jnp.bfloat16)
        up_w = buf13[slot, 1].astype(jnp.bfloat16)
        gate = jax.lax.dot_general(xb, gate_w, dims,
                                   preferred_element_type=jnp.float32)
        up = jax.lax.dot_general(xb, up_w, dims,
                                 preferred_element_type=jnp.float32)
        act = gate * jax.nn.sigmoid(gate) * up
        scale = jnp.sum(jnp.where(eidx == e, wte_ref[...], 0.0), axis=-1,
                        keepdims=True)
        actb = (act * scale).astype(jnp.bfloat16)
        pltpu.make_async_copy(
            w2_hbm.at[e, :, pl.ds(i * _BI, _BI)],
            buf2.at[slot], sem2.at[slot]).wait()
        w2b = buf2[slot].astype(jnp.bfloat16)
        out_ref[...] += jax.lax.dot_general(
            actb, w2b, dims, preferred_element_type=jnp.float32)

        @pl.when(c < _NCHUNKS - _NBUF)
        def _():
            nc = c + _NBUF
            _issue(w13_hbm, w2_hbm, buf13, buf2, sem13, sem2,
                   nc // 2, jax.lax.rem(nc, 2), slot)
        return ()

    jax.lax.fori_loop(0, _NCHUNKS, chunk_body, (), unroll=False)


def kernel(x, router_logits, w13, w2):
    w13r = w13.reshape(_NUM_EXPERTS, 2, _INTER, _HIDDEN)
    return pl.pallas_call(
        _moe_body,
        in_specs=[
            pl.BlockSpec(memory_space=pltpu.VMEM),
            pl.BlockSpec(memory_space=pltpu.VMEM),
            pl.BlockSpec(memory_space=pl.ANY),
            pl.BlockSpec(memory_space=pl.ANY),
        ],
        out_specs=pl.BlockSpec(memory_space=pltpu.VMEM),
        out_shape=jax.ShapeDtypeStruct((_NUM_TOKENS, _HIDDEN), jnp.float32),
        scratch_shapes=[
            pltpu.VMEM((_NUM_TOKENS, _NUM_EXPERTS), jnp.float32),
            pltpu.VMEM((_NBUF, 2, _BI, _HIDDEN), jnp.float32),
            pltpu.VMEM((_NBUF, _HIDDEN, _BI), jnp.float32),
            pltpu.SemaphoreType.DMA((_NBUF,)),
            pltpu.SemaphoreType.DMA((_NBUF,)),
        ],
    )(x, router_logits, w13r, w2)
